# baseline (speedup 1.0000x reference)
import functools

import jax
import jax.numpy as jnp
from jax.experimental import pallas as pl
from jax.experimental.pallas import tpu as pltpu

_H = 64
_BE = 4000


def _leaky(v):
    return jnp.where(v > 0, v, 0.2 * v)


def _edge_tc_body(s1_ref, c_ref, cd_ref, w1r_ref, b1_ref, w2_ref, b2_ref,
                  cw1_ref, cb1_ref, cw2_ref, m_ref, t4_ref):
    s1 = s1_ref[...]
    c = c_ref[...]
    cd = cd_ref[...]
    rad = cd[:, 3:4]
    m1 = _leaky(s1 + c + rad * w1r_ref[...] + b1_ref[...])
    m = _leaky(jnp.dot(m1, w2_ref[...], preferred_element_type=jnp.float32)
               + b2_ref[...])
    cm1 = _leaky(jnp.dot(m, cw1_ref[...], preferred_element_type=jnp.float32)
                 + cb1_ref[...])
    cmv = jnp.sum(cm1 * cw2_ref[...], axis=1, keepdims=True)
    trans = jnp.clip(cd[:, 0:3] * cmv, -100.0, 100.0)
    m_ref[...] = m
    t4_ref[...] = jnp.concatenate([trans, jnp.ones_like(cmv)], axis=1)


def _edge_stage(s1, c, cd, w1r, b1, w2, b2, cw1, cb1, cw2):
    e = s1.shape[0]
    grid = (e // _BE,)
    full = lambda shape: pl.BlockSpec(shape, lambda i: (0, 0))
    return pl.pallas_call(
        _edge_tc_body,
        grid=grid,
        in_specs=[
            pl.BlockSpec((_BE, _H), lambda i: (i, 0)),
            pl.BlockSpec((_BE, _H), lambda i: (i, 0)),
            pl.BlockSpec((_BE, 4), lambda i: (i, 0)),
            full((1, _H)),
            full((1, _H)),
            full((_H, _H)),
            full((1, _H)),
            full((_H, _H)),
            full((1, _H)),
            full((1, _H)),
        ],
        out_specs=[
            pl.BlockSpec((_BE, _H), lambda i: (i, 0)),
            pl.BlockSpec((_BE, 4), lambda i: (i, 0)),
        ],
        out_shape=[
            jax.ShapeDtypeStruct((e, _H), jnp.float32),
            jax.ShapeDtypeStruct((e, 4), jnp.float32),
        ],
    )(s1, c, cd, w1r, b1, w2, b2, cw1, cb1, cw2)


def kernel(h, x, edges, edge_attr, emb_w, emb_b, edge_w1, edge_b1, edge_w2,
           edge_b2, node_w1, node_b1, node_w2, node_b2, coord_w1, coord_b1,
           coord_w2):
    n = h.shape[0]
    num_layers = edge_w1.shape[0]
    row = edges[0]
    col = edges[1]

    h = h @ emb_w + emb_b

    c_all = [edge_attr @ edge_w1[i][2 * _H + 1:] for i in range(num_layers)]

    for i in range(num_layers):
        w1 = edge_w1[i]
        a = h @ w1[:_H]
        b = h @ w1[_H:2 * _H]
        s1 = jnp.take(a, row, axis=0) + jnp.take(b, col, axis=0)
        xr = jnp.take(x, row, axis=0)
        xc = jnp.take(x, col, axis=0)
        cd3 = xr - xc
        rad = jnp.sum(cd3 * cd3, axis=1, keepdims=True)
        cd = jnp.concatenate([cd3, rad], axis=1)
        m, t4 = _edge_stage(
            s1, c_all[i], cd,
            w1[2 * _H:2 * _H + 1], edge_b1[i][None, :], edge_w2[i],
            edge_b2[i][None, :], coord_w1[i], coord_b1[i][None, :],
            coord_w2[i][:, 0][None, :])
        agg4 = jax.ops.segment_sum(t4, row, num_segments=n)
        cnt = jnp.clip(agg4[:, 3:4], 1.0, None)
        x = x + agg4[:, 0:3] / cnt
        if i + 1 < num_layers:
            nagg = jax.ops.segment_sum(m, row, num_segments=n)
            o = _leaky(h @ node_w1[i][:_H] + nagg @ node_w1[i][_H:]
                       + node_b1[i])
            o = o @ node_w2[i] + node_b2[i]
            h = h + o
    return x


# speedup vs baseline: 1.7830x; 1.7830x over previous
import functools

import jax
import jax.numpy as jnp
from jax import lax
from jax.experimental import pallas as pl
from jax.experimental.pallas import tpu as pltpu
from jax.experimental.pallas import tpu_sc as plsc

_H = 64
_K = 128
_NW = 32
_NH = 25600
_RPT = 1600
_NPAD = 2 * _NH
_EPAD = 819200
_CPW = _EPAD // (_NW * _K)
_EPT = _EPAD // 16
_CPT = _EPT // _K
_BE = 2048
_BN = 512
_H2 = 32


def _leaky(v):
    return jnp.where(v > 0, v, 0.2 * v)


def _mesh():
    return plsc.VectorSubcoreMesh(core_axis_name="c", subcore_axis_name="s")


def _gather_body(tr_hbm, tc_hbm, rg_hbm, cg_hbm, scd_hbm,
                 ridx, cidx, rbuf, cbuf, sbuf, sem):
    cid = lax.axis_index("c")
    sid = lax.axis_index("s")
    wid = sid * 2 + cid

    def chunk(j, carry):
        base = pl.multiple_of((wid * _CPW + j) * _K, _K)
        pltpu.sync_copy(rg_hbm.at[pl.ds(base, _K)], ridx)
        pltpu.sync_copy(cg_hbm.at[pl.ds(base, _K)], cidx)
        d1 = pltpu.async_copy(tr_hbm.at[ridx], rbuf, sem)
        d2 = pltpu.async_copy(tc_hbm.at[cidx], cbuf, sem)
        d1.wait()
        d2.wait()

        def rowfn(r, c2):
            for k in range(8):
                sl = pl.ds(k * 16, 16)
                sbuf[r, sl] = rbuf[r, sl] + cbuf[r, sl]
            return c2

        lax.fori_loop(0, _K, rowfn, 0, unroll=2)
        pltpu.sync_copy(sbuf, scd_hbm.at[pl.ds(base, _K)])
        return carry

    lax.fori_loop(0, _CPW, chunk, 0)


def _sc_gather(tr, tc, rowg, colg):
    f = pl.kernel(
        _gather_body,
        out_type=jax.ShapeDtypeStruct((_EPAD, 128), jnp.float32),
        mesh=_mesh(),
        scratch_types=[
            pltpu.VMEM((_K,), jnp.int32),
            pltpu.VMEM((_K,), jnp.int32),
            pltpu.VMEM((_K, 128), jnp.float32),
            pltpu.VMEM((_K, 128), jnp.float32),
            pltpu.VMEM((_K, 128), jnp.float32),
            pltpu.SemaphoreType.DMA,
        ],
    )
    return f(tr, tc, rowg, colg)


_ZRM = 160
_ZRT = 320


_OFFS = list(range(0, _RPT - _K, _K)) + [_RPT - _K]


def _scatter_body_factory(im, it):
    w = _H2 if im else 16

    def body(*refs):
        it_refs = iter(refs)
        maw_hbm = next(it_refs) if im else None
        tw_hbm = next(it_refs) if it else None
        rs_hbm = next(it_refs)
        out_hbm = next(it_refs)
        idxb = next(it_refs)
        lidxb = next(it_refs)
        stage = next(it_refs)
        sbuf = next(it_refs)
        vbuf = next(it_refs)
        pbuf = next(it_refs)
        idxseq = next(it_refs)
        sacc = next(it_refs)

        cid = lax.axis_index("c")
        sid = lax.axis_index("s")
        base_node = cid * _NH
        trow = sid * _RPT

        def vfill(r, carry):
            for k in range(w // 16):
                vbuf[r, pl.ds(k * 16, 16)] = jnp.zeros((16,), jnp.float32)
            return carry

        lax.fori_loop(0, _K, vfill, 0)

        def seqfill(off):
            for v in range(_K // 16):
                idxseq[pl.ds(v * 16, 16)] = (
                    lax.iota(jnp.int32, 16) + (trow + off + v * 16))

        for off in _OFFS:
            seqfill(off)
            pltpu.sync_copy(vbuf, sacc.at[idxseq])
        plsc.subcore_barrier()

        def chunk(j, carry):
            base = pl.multiple_of(sid * _EPT + j * _K, _K)
            pltpu.sync_copy(rs_hbm.at[pl.ds(base, _K)], idxb)
            for v in range(_K // 16):
                sl = pl.ds(v * 16, 16)
                g = idxb[sl] - base_node
                ok = (g >= 0) & (g < _NH)
                lidxb[sl] = jnp.where(ok, g, _NH)
            if im:
                wbase = pl.multiple_of(sid * (_EPT // 4) + j * (_K // 4),
                                       _K // 4)
                pltpu.sync_copy(maw_hbm.at[pl.ds(wbase, _K // 4)], stage)

                def unpackm(q, c2):
                    for k in range(4):
                        for h2 in range(2):
                            sbuf[q * 4 + k, pl.ds(h2 * 16, 16)] = (
                                stage[q, pl.ds(k * 32 + h2 * 16, 16)])
                    return c2

                lax.fori_loop(0, _K // 4, unpackm, 0)
            else:
                tb = pl.multiple_of(sid * (_EPT // 8) + j * (_K // 8), _K // 8)
                pltpu.sync_copy(tw_hbm.at[pl.ds(tb, _K // 8)], stage)

                def unpackt(q, c2):
                    for k in range(8):
                        sbuf[q * 8 + k, pl.ds(0, 16)] = (
                            stage[q, pl.ds(k * 16, 16)])
                    return c2

                lax.fori_loop(0, _K // 8, unpackt, 0)
            pltpu.sync_copy(sbuf, sacc.at[lidxb], add=True)
            return carry

        lax.fori_loop(0, _CPT, chunk, 0)
        plsc.subcore_barrier()

        for off in _OFFS:
            for g2 in range(_K // 16):
                vals = lax.iota(jnp.int32, 16) + (trow + off + g2 * 16)
                pltpu.sync_copy(sacc.at[vals], vbuf.at[pl.ds(g2 * 16, 16)])
            if im:
                def packm(q, c2):
                    for k in range(4):
                        for h2 in range(2):
                            pbuf[q, pl.ds(k * 32 + h2 * 16, 16)] = (
                                vbuf[q * 4 + k, pl.ds(h2 * 16, 16)])
                    return c2

                lax.fori_loop(0, _K // 4, packm, 0)
                pltpu.sync_copy(
                    pbuf.at[pl.ds(0, _K // 4)],
                    out_hbm.at[pl.ds(
                        pl.multiple_of((base_node + trow + off) // 4,
                                       _K // 4), _K // 4)])
            else:
                def packt(q, c2):
                    for k in range(8):
                        pbuf[q, pl.ds(k * 16, 16)] = (
                            vbuf[q * 8 + k, pl.ds(0, 16)])
                    return c2

                lax.fori_loop(0, _K // 8, packt, 0)
                pltpu.sync_copy(
                    pbuf.at[pl.ds(0, _K // 8)],
                    out_hbm.at[pl.ds(
                        pl.multiple_of((base_node + trow + off) // 8,
                                       _K // 8), _K // 8)])

    return body


def _sc_scatter(maw, tw, rows):
    im = maw is not None
    w = _H2 if im else 16
    out_type = jax.ShapeDtypeStruct((_NPAD * w // 128, 128), jnp.float32)
    scratch = [
        pltpu.VMEM((_K,), jnp.int32),
        pltpu.VMEM((_K,), jnp.int32),
        pltpu.VMEM((_K * w // 128, 128), jnp.float32),
        pltpu.VMEM((_K, w), jnp.float32),
        pltpu.VMEM((_K, w), jnp.float32),
        pltpu.VMEM((_K * w // 128, 128), jnp.float32),
        pltpu.VMEM((_K,), jnp.int32),
        pltpu.VMEM_SHARED((_NH + 8, w), jnp.float32),
    ]
    f = pl.kernel(
        _scatter_body_factory(im, tw is not None),
        out_type=[out_type],
        mesh=_mesh(),
        scratch_types=scratch,
    )
    args = [a for a in (maw, tw, rows) if a is not None]
    return f(*args)[0]


def _edge_tc_body(scd_ref, c_ref, w1r_ref, b1_ref, w2_ref,
                  b2_ref, cw1_ref, cb1_ref, cw2_ref, mt_ref):
    scd = scd_ref[...]
    s1 = scd[:, 0:_H]
    cd3 = scd[:, _H:_H + 3]
    rad = jnp.sum(cd3 * cd3, axis=1, keepdims=True)
    m1 = _leaky(s1 + c_ref[...] + rad * w1r_ref[...] + b1_ref[...])
    m = _leaky(jnp.dot(m1, w2_ref[...], preferred_element_type=jnp.float32)
               + b2_ref[...])
    cm1 = _leaky(jnp.dot(m, cw1_ref[...], preferred_element_type=jnp.float32)
                 + cb1_ref[...])
    cmv = jnp.sum(cm1 * cw2_ref[...], axis=1, keepdims=True)
    trans = jnp.clip(cd3 * cmv, -100.0, 100.0)
    mt_ref[...] = jnp.concatenate(
        [m, trans, jnp.ones_like(cmv),
         jnp.zeros((trans.shape[0], 60), jnp.float32)], axis=1)


def _edge_stage(scd, c, w1r, b1, w2, b2, cw1, cb1, cw2):
    grid = (_EPAD // _BE,)
    full = lambda shape: pl.BlockSpec(shape, lambda i: (0, 0))
    return pl.pallas_call(
        _edge_tc_body,
        grid=grid,
        in_specs=[
            pl.BlockSpec((_BE, 128), lambda i: (i, 0)),
            pl.BlockSpec((_BE, _H), lambda i: (i, 0)),
            full((1, _H)),
            full((1, _H)),
            full((_H, _H)),
            full((1, _H)),
            full((_H, _H)),
            full((1, _H)),
            full((1, _H)),
        ],
        out_specs=pl.BlockSpec((_BE, 128), lambda i: (i, 0)),
        out_shape=jax.ShapeDtypeStruct((_EPAD, 128), jnp.float32),
    )(scd, c, w1r, b1, w2, b2, cw1, cb1, cw2)


def _tables(a, b, xpc):
    xp16 = xpc[:, 0:16]
    pad = jnp.zeros((a.shape[0], 48), jnp.float32)
    tr = jnp.concatenate([a, xp16, pad], axis=1)
    tc = jnp.concatenate([b, -xp16, pad], axis=1)
    return tr, tc


def _prep0_body(h_ref, xp_ref, embw_ref, embb_ref, w1a_ref, w1b_ref,
                h0_ref, tr_ref, tc_ref):
    h0 = (jnp.dot(h_ref[...], embw_ref[...], preferred_element_type=jnp.float32)
          + embb_ref[...])
    h0_ref[...] = h0
    a = jnp.dot(h0, w1a_ref[...], preferred_element_type=jnp.float32)
    b = jnp.dot(h0, w1b_ref[...], preferred_element_type=jnp.float32)
    tr, tc = _tables(a, b, xp_ref[...])
    tr_ref[...] = tr
    tc_ref[...] = tc


def _prep0(h128, xp, emb_w, emb_b, w1a, w1b):
    grid = (_NPAD // _BN,)
    full = lambda shape: pl.BlockSpec(shape, lambda i: (0, 0))
    d_in = h128.shape[1]
    return pl.pallas_call(
        _prep0_body,
        grid=grid,
        in_specs=[
            pl.BlockSpec((_BN, d_in), lambda i: (i, 0)),
            pl.BlockSpec((_BN, 128), lambda i: (i, 0)),
            full((d_in, _H)),
            full((1, _H)),
            full((_H, _H)),
            full((_H, _H)),
        ],
        out_specs=[
            pl.BlockSpec((_BN, _H), lambda i: (i, 0)),
            pl.BlockSpec((_BN, 128), lambda i: (i, 0)),
            pl.BlockSpec((_BN, 128), lambda i: (i, 0)),
        ],
        out_shape=[
            jax.ShapeDtypeStruct((_NPAD, _H), jnp.float32),
            jax.ShapeDtypeStruct((_NPAD, 128), jnp.float32),
            jax.ShapeDtypeStruct((_NPAD, 128), jnp.float32),
        ],
    )(h128, xp, emb_w, emb_b, w1a, w1b)


def _node_body(h_ref, naggt_ref, xp_ref, nw1a_ref, nw1b_ref, nb1_ref,
               nw2_ref, nb2_ref, w1a_ref, w1b_ref,
               hn_ref, trn_ref, tcn_ref, xpn_ref):
    naggt = naggt_ref[...]
    cnt = jnp.clip(naggt[:, _H + 3:_H + 4], 1.0, None)
    xn3 = xp_ref[:, 0:3] + naggt[:, _H:_H + 3] / cnt
    xpn = jnp.concatenate(
        [xn3, jnp.zeros((xn3.shape[0], 125), jnp.float32)], axis=1)
    xpn_ref[...] = xpn
    h = h_ref[...]
    o = _leaky(jnp.dot(h, nw1a_ref[...], preferred_element_type=jnp.float32)
               + jnp.dot(naggt[:, 0:_H], nw1b_ref[...],
                         preferred_element_type=jnp.float32)
               + nb1_ref[...])
    o = jnp.dot(o, nw2_ref[...], preferred_element_type=jnp.float32) + nb2_ref[...]
    hn = h + o
    hn_ref[...] = hn
    a = jnp.dot(hn, w1a_ref[...], preferred_element_type=jnp.float32)
    b = jnp.dot(hn, w1b_ref[...], preferred_element_type=jnp.float32)
    tr, tc = _tables(a, b, xpn)
    trn_ref[...] = tr
    tcn_ref[...] = tc


def _node_stage(h, naggt, xp, nw1a, nw1b, nb1, nw2, nb2, w1a, w1b):
    grid = (_NPAD // _BN,)
    full = lambda shape: pl.BlockSpec(shape, lambda i: (0, 0))
    return pl.pallas_call(
        _node_body,
        grid=grid,
        in_specs=[
            pl.BlockSpec((_BN, _H), lambda i: (i, 0)),
            pl.BlockSpec((_BN, 128), lambda i: (i, 0)),
            pl.BlockSpec((_BN, 128), lambda i: (i, 0)),
            full((_H, _H)),
            full((_H, _H)),
            full((1, _H)),
            full((_H, _H)),
            full((1, _H)),
            full((_H, _H)),
            full((_H, _H)),
        ],
        out_specs=[
            pl.BlockSpec((_BN, _H), lambda i: (i, 0)),
            pl.BlockSpec((_BN, 128), lambda i: (i, 0)),
            pl.BlockSpec((_BN, 128), lambda i: (i, 0)),
            pl.BlockSpec((_BN, 128), lambda i: (i, 0)),
        ],
        out_shape=[
            jax.ShapeDtypeStruct((_NPAD, _H), jnp.float32),
            jax.ShapeDtypeStruct((_NPAD, 128), jnp.float32),
            jax.ShapeDtypeStruct((_NPAD, 128), jnp.float32),
            jax.ShapeDtypeStruct((_NPAD, 128), jnp.float32),
        ],
    )(h, naggt, xp, nw1a, nw1b, nb1, nw2, nb2, w1a, w1b)


def _finalx_body(naggt_ref, xp_ref, xo_ref):
    naggt = naggt_ref[...]
    cnt = jnp.clip(naggt[:, _H + 3:_H + 4], 1.0, None)
    xn3 = xp_ref[:, 0:3] + naggt[:, _H:_H + 3] / cnt
    xo_ref[...] = jnp.concatenate(
        [xn3, jnp.zeros((xn3.shape[0], 125), jnp.float32)], axis=1)


def _finalx_stage(naggt, xp):
    grid = (_NPAD // _BN,)
    return pl.pallas_call(
        _finalx_body,
        grid=grid,
        in_specs=[
            pl.BlockSpec((_BN, 128), lambda i: (i, 0)),
            pl.BlockSpec((_BN, 128), lambda i: (i, 0)),
        ],
        out_specs=pl.BlockSpec((_BN, 128), lambda i: (i, 0)),
        out_shape=jax.ShapeDtypeStruct((_NPAD, 128), jnp.float32),
    )(naggt, xp)


def kernel(h, x, edges, edge_attr, emb_w, emb_b, edge_w1, edge_b1, edge_w2,
           edge_b2, node_w1, node_b1, node_w2, node_b2, coord_w1, coord_b1,
           coord_w2):
    n = h.shape[0]
    e = edges.shape[1]
    num_layers = edge_w1.shape[0]
    row = edges[0]
    col = edges[1]

    rowg = jnp.pad(row, (0, _EPAD - e))
    colg = jnp.pad(col, (0, _EPAD - e))
    rows = jnp.pad(row, (0, _EPAD - e), constant_values=_NPAD - 1)
    ea_pad = jnp.pad(edge_attr, ((0, _EPAD - e), (0, 0)))
    h_pad = jnp.pad(h, ((0, _NPAD - n), (0, 0)))
    xp = jnp.pad(x, ((0, _NPAD - n), (0, 125)))

    hcur, tr, tc = _prep0(h_pad, xp, emb_w, emb_b[None, :],
                          edge_w1[0][:_H], edge_w1[0][_H:2 * _H])


    for i in range(num_layers):
        w1 = edge_w1[i]
        scd = _sc_gather(tr, tc, rowg, colg)
        mt = _edge_stage(
            scd, ea_pad @ w1[2 * _H + 1:], w1[2 * _H:2 * _H + 1],
            edge_b1[i][None, :], edge_w2[i], edge_b2[i][None, :],
            coord_w1[i], coord_b1[i][None, :], coord_w2[i][:, 0][None, :])
        naggt = jax.ops.segment_sum(mt, rows, num_segments=_NPAD)
        if i + 1 < num_layers:
            w1n = edge_w1[i + 1]
            hcur, tr, tc, xp = _node_stage(
                hcur, naggt, xp, node_w1[i][:_H], node_w1[i][_H:],
                node_b1[i][None, :], node_w2[i], node_b2[i][None, :],
                w1n[:_H], w1n[_H:2 * _H])
        else:
            xo = _finalx_stage(naggt, xp)
    return xo[:n, :3]


# speedup vs baseline: 1.7837x; 1.0004x over previous
import functools

import jax
import jax.numpy as jnp
from jax import lax
from jax.experimental import pallas as pl
from jax.experimental.pallas import tpu as pltpu
from jax.experimental.pallas import tpu_sc as plsc

_H = 64
_K = 128
_NW = 32
_NH = 25600
_RPT = 1600
_NPAD = 2 * _NH
_EPAD = 819200
_CPW = _EPAD // (_NW * _K)
_EPT = _EPAD // 16
_CPT = _EPT // _K
_BE = 2048
_BN = 512
_H2 = 32


def _leaky(v):
    return jnp.where(v > 0, v, 0.2 * v)


def _mesh():
    return plsc.VectorSubcoreMesh(core_axis_name="c", subcore_axis_name="s")


def _gather_body(tr_hbm, tc_hbm, rg_hbm, cg_hbm, scd_hbm,
                 ridx, cidx, rbuf, cbuf, sbuf, sem):
    cid = lax.axis_index("c")
    sid = lax.axis_index("s")
    wid = sid * 2 + cid

    def chunk(j, carry):
        base = pl.multiple_of((wid * _CPW + j) * _K, _K)
        pltpu.sync_copy(rg_hbm.at[pl.ds(base, _K)], ridx)
        pltpu.sync_copy(cg_hbm.at[pl.ds(base, _K)], cidx)
        d1 = pltpu.async_copy(tr_hbm.at[ridx], rbuf, sem)
        d2 = pltpu.async_copy(tc_hbm.at[cidx], cbuf, sem)
        d1.wait()
        d2.wait()

        def rowfn(r, c2):
            for k in range(8):
                sl = pl.ds(k * 16, 16)
                sbuf[r, sl] = rbuf[r, sl] + cbuf[r, sl]
            return c2

        lax.fori_loop(0, _K, rowfn, 0, unroll=2)
        pltpu.sync_copy(sbuf, scd_hbm.at[pl.ds(base, _K)])
        return carry

    lax.fori_loop(0, _CPW, chunk, 0)


def _sc_gather(tr, tc, rowg, colg):
    f = pl.kernel(
        _gather_body,
        out_type=jax.ShapeDtypeStruct((_EPAD, 128), jnp.float32),
        mesh=_mesh(),
        scratch_types=[
            pltpu.VMEM((_K,), jnp.int32),
            pltpu.VMEM((_K,), jnp.int32),
            pltpu.VMEM((_K, 128), jnp.float32),
            pltpu.VMEM((_K, 128), jnp.float32),
            pltpu.VMEM((_K, 128), jnp.float32),
            pltpu.SemaphoreType.DMA,
        ],
    )
    return f(tr, tc, rowg, colg)


def _edge_tc_body(scd_ref, c_ref, w1r_ref, b1_ref, w2_ref,
                  b2_ref, cw1_ref, cb1_ref, cw2_ref, mt_ref):
    scd = scd_ref[...]
    s1 = scd[:, 0:_H]
    cd3 = scd[:, _H:_H + 3]
    rad = jnp.sum(cd3 * cd3, axis=1, keepdims=True)
    m1 = _leaky(s1 + c_ref[...] + rad * w1r_ref[...] + b1_ref[...])
    m = _leaky(jnp.dot(m1, w2_ref[...], preferred_element_type=jnp.float32)
               + b2_ref[...])
    cm1 = _leaky(jnp.dot(m, cw1_ref[...], preferred_element_type=jnp.float32)
                 + cb1_ref[...])
    cmv = jnp.sum(cm1 * cw2_ref[...], axis=1, keepdims=True)
    trans = jnp.clip(cd3 * cmv, -100.0, 100.0)
    mt_ref[...] = jnp.concatenate(
        [m, trans, jnp.ones_like(cmv),
         jnp.zeros((trans.shape[0], 60), jnp.float32)], axis=1)


def _edge_stage(scd, c, w1r, b1, w2, b2, cw1, cb1, cw2):
    grid = (_EPAD // _BE,)
    full = lambda shape: pl.BlockSpec(shape, lambda i: (0, 0))
    return pl.pallas_call(
        _edge_tc_body,
        grid=grid,
        in_specs=[
            pl.BlockSpec((_BE, 128), lambda i: (i, 0)),
            pl.BlockSpec((_BE, _H), lambda i: (i, 0)),
            full((1, _H)),
            full((1, _H)),
            full((_H, _H)),
            full((1, _H)),
            full((_H, _H)),
            full((1, _H)),
            full((1, _H)),
        ],
        out_specs=pl.BlockSpec((_BE, 128), lambda i: (i, 0)),
        out_shape=jax.ShapeDtypeStruct((_EPAD, 128), jnp.float32),
    )(scd, c, w1r, b1, w2, b2, cw1, cb1, cw2)


def _tables(a, b, xpc):
    xp16 = xpc[:, 0:16]
    pad = jnp.zeros((a.shape[0], 48), jnp.float32)
    tr = jnp.concatenate([a, xp16, pad], axis=1)
    tc = jnp.concatenate([b, -xp16, pad], axis=1)
    return tr, tc


def _prep0_body(h_ref, xp_ref, embw_ref, embb_ref, w1a_ref, w1b_ref,
                h0_ref, tr_ref, tc_ref):
    h0 = (jnp.dot(h_ref[...], embw_ref[...], preferred_element_type=jnp.float32)
          + embb_ref[...])
    h0_ref[...] = h0
    a = jnp.dot(h0, w1a_ref[...], preferred_element_type=jnp.float32)
    b = jnp.dot(h0, w1b_ref[...], preferred_element_type=jnp.float32)
    tr, tc = _tables(a, b, xp_ref[...])
    tr_ref[...] = tr
    tc_ref[...] = tc


def _prep0(h128, xp, emb_w, emb_b, w1a, w1b):
    grid = (_NPAD // _BN,)
    full = lambda shape: pl.BlockSpec(shape, lambda i: (0, 0))
    d_in = h128.shape[1]
    return pl.pallas_call(
        _prep0_body,
        grid=grid,
        in_specs=[
            pl.BlockSpec((_BN, d_in), lambda i: (i, 0)),
            pl.BlockSpec((_BN, 128), lambda i: (i, 0)),
            full((d_in, _H)),
            full((1, _H)),
            full((_H, _H)),
            full((_H, _H)),
        ],
        out_specs=[
            pl.BlockSpec((_BN, _H), lambda i: (i, 0)),
            pl.BlockSpec((_BN, 128), lambda i: (i, 0)),
            pl.BlockSpec((_BN, 128), lambda i: (i, 0)),
        ],
        out_shape=[
            jax.ShapeDtypeStruct((_NPAD, _H), jnp.float32),
            jax.ShapeDtypeStruct((_NPAD, 128), jnp.float32),
            jax.ShapeDtypeStruct((_NPAD, 128), jnp.float32),
        ],
    )(h128, xp, emb_w, emb_b, w1a, w1b)


def _node_body(h_ref, naggt_ref, xp_ref, nw1a_ref, nw1b_ref, nb1_ref,
               nw2_ref, nb2_ref, w1a_ref, w1b_ref,
               hn_ref, trn_ref, tcn_ref, xpn_ref):
    naggt = naggt_ref[...]
    cnt = jnp.clip(naggt[:, _H + 3:_H + 4], 1.0, None)
    xn3 = xp_ref[:, 0:3] + naggt[:, _H:_H + 3] / cnt
    xpn = jnp.concatenate(
        [xn3, jnp.zeros((xn3.shape[0], 125), jnp.float32)], axis=1)
    xpn_ref[...] = xpn
    h = h_ref[...]
    o = _leaky(jnp.dot(h, nw1a_ref[...], preferred_element_type=jnp.float32)
               + jnp.dot(naggt[:, 0:_H], nw1b_ref[...],
                         preferred_element_type=jnp.float32)
               + nb1_ref[...])
    o = jnp.dot(o, nw2_ref[...], preferred_element_type=jnp.float32) + nb2_ref[...]
    hn = h + o
    hn_ref[...] = hn
    a = jnp.dot(hn, w1a_ref[...], preferred_element_type=jnp.float32)
    b = jnp.dot(hn, w1b_ref[...], preferred_element_type=jnp.float32)
    tr, tc = _tables(a, b, xpn)
    trn_ref[...] = tr
    tcn_ref[...] = tc


def _node_stage(h, naggt, xp, nw1a, nw1b, nb1, nw2, nb2, w1a, w1b):
    grid = (_NPAD // _BN,)
    full = lambda shape: pl.BlockSpec(shape, lambda i: (0, 0))
    return pl.pallas_call(
        _node_body,
        grid=grid,
        in_specs=[
            pl.BlockSpec((_BN, _H), lambda i: (i, 0)),
            pl.BlockSpec((_BN, 128), lambda i: (i, 0)),
            pl.BlockSpec((_BN, 128), lambda i: (i, 0)),
            full((_H, _H)),
            full((_H, _H)),
            full((1, _H)),
            full((_H, _H)),
            full((1, _H)),
            full((_H, _H)),
            full((_H, _H)),
        ],
        out_specs=[
            pl.BlockSpec((_BN, _H), lambda i: (i, 0)),
            pl.BlockSpec((_BN, 128), lambda i: (i, 0)),
            pl.BlockSpec((_BN, 128), lambda i: (i, 0)),
            pl.BlockSpec((_BN, 128), lambda i: (i, 0)),
        ],
        out_shape=[
            jax.ShapeDtypeStruct((_NPAD, _H), jnp.float32),
            jax.ShapeDtypeStruct((_NPAD, 128), jnp.float32),
            jax.ShapeDtypeStruct((_NPAD, 128), jnp.float32),
            jax.ShapeDtypeStruct((_NPAD, 128), jnp.float32),
        ],
    )(h, naggt, xp, nw1a, nw1b, nb1, nw2, nb2, w1a, w1b)


def _finalx_body(naggt_ref, xp_ref, xo_ref):
    naggt = naggt_ref[...]
    cnt = jnp.clip(naggt[:, _H + 3:_H + 4], 1.0, None)
    xn3 = xp_ref[:, 0:3] + naggt[:, _H:_H + 3] / cnt
    xo_ref[...] = jnp.concatenate(
        [xn3, jnp.zeros((xn3.shape[0], 125), jnp.float32)], axis=1)


def _finalx_stage(naggt, xp):
    grid = (_NPAD // _BN,)
    return pl.pallas_call(
        _finalx_body,
        grid=grid,
        in_specs=[
            pl.BlockSpec((_BN, 128), lambda i: (i, 0)),
            pl.BlockSpec((_BN, 128), lambda i: (i, 0)),
        ],
        out_specs=pl.BlockSpec((_BN, 128), lambda i: (i, 0)),
        out_shape=jax.ShapeDtypeStruct((_NPAD, 128), jnp.float32),
    )(naggt, xp)


def kernel(h, x, edges, edge_attr, emb_w, emb_b, edge_w1, edge_b1, edge_w2,
           edge_b2, node_w1, node_b1, node_w2, node_b2, coord_w1, coord_b1,
           coord_w2):
    n = h.shape[0]
    e = edges.shape[1]
    num_layers = edge_w1.shape[0]
    row = edges[0]
    col = edges[1]

    rowg = jnp.pad(row, (0, _EPAD - e))
    colg = jnp.pad(col, (0, _EPAD - e))
    rows = jnp.pad(row, (0, _EPAD - e), constant_values=_NPAD - 1)
    ea_pad = jnp.pad(edge_attr, ((0, _EPAD - e), (0, 0)))
    h_pad = jnp.pad(h, ((0, _NPAD - n), (0, 0)))
    xp = jnp.pad(x, ((0, _NPAD - n), (0, 125)))

    hcur, tr, tc = _prep0(h_pad, xp, emb_w, emb_b[None, :],
                          edge_w1[0][:_H], edge_w1[0][_H:2 * _H])


    for i in range(num_layers):
        w1 = edge_w1[i]
        scd = _sc_gather(tr, tc, rowg, colg)
        mt = _edge_stage(
            scd, ea_pad @ w1[2 * _H + 1:], w1[2 * _H:2 * _H + 1],
            edge_b1[i][None, :], edge_w2[i], edge_b2[i][None, :],
            coord_w1[i], coord_b1[i][None, :], coord_w2[i][:, 0][None, :])
        naggt = jax.ops.segment_sum(mt, rows, num_segments=_NPAD)
        if i + 1 < num_layers:
            w1n = edge_w1[i + 1]
            hcur, tr, tc, xp = _node_stage(
                hcur, naggt, xp, node_w1[i][:_H], node_w1[i][_H:],
                node_b1[i][None, :], node_w2[i], node_b2[i][None, :],
                w1n[:_H], w1n[_H:2 * _H])
        else:
            xo = _finalx_stage(naggt, xp)
    return xo[:n, :3]
